# baseline (speedup 1.0000x reference)
import jax
import jax.numpy as jnp

VOXEL_DIM = (96, 96, 48)
VOXEL_SIZE = 0.04


def kernel(projection, feature, depth, origin):
    B, C, H, W = feature.shape
    X, Y, Z = VOXEL_DIM
    HW = H * W
    NV = X * Y * Z

    bottom = jnp.broadcast_to(
        jnp.array([0.0, 0.0, 0.0, 1.0], dtype=projection.dtype).reshape(1, 1, 4),
        (B, 1, 4))
    new_projection = jnp.concatenate([projection, bottom], axis=1)
    inv = jnp.linalg.inv(new_projection)

    d = depth.reshape(B, HW)
    hw = jnp.arange(HW, dtype=jnp.int32)
    u = jnp.broadcast_to((hw % W).astype(jnp.float32)[None], (B, HW))
    v = jnp.broadcast_to((hw // W).astype(jnp.float32)[None], (B, HW))

    def b16(x):
        return jax.lax.optimization_barrier(x.astype(jnp.bfloat16)).astype(jnp.float32)

    ud = b16(u * d)
    vd = b16(v * d)
    db = b16(d)

    def wrow(i):
        return (b16(inv[:, i, 0:1]) * ud + b16(inv[:, i, 1:2]) * vd
                + b16(inv[:, i, 2:3]) * db + b16(inv[:, i, 3:4]))

    wx, wy, wz = wrow(0), wrow(1), wrow(2)
    vx = jnp.round((wx - origin[:, 0:1]) / VOXEL_SIZE).astype(jnp.int32)
    vy = jnp.round((wy - origin[:, 1:2]) / VOXEL_SIZE).astype(jnp.int32)
    vz = jnp.round((wz - origin[:, 2:3]) / VOXEL_SIZE).astype(jnp.int32)

    mask = ((d > 0) & (vx >= 0) & (vx < X) & (vy >= 0) & (vy < Y)
            & (vz >= 0) & (vz < Z))
    flat = vx * (Y * Z) + vy * Z + vz
    gidx = jnp.where(mask, flat + jnp.arange(B, dtype=jnp.int32)[:, None] * (NV + 1),
                     jnp.arange(B, dtype=jnp.int32)[:, None] * (NV + 1) + NV)
    gidx = gidx.reshape(-1)

    pid = jnp.arange(B * HW, dtype=jnp.int32)
    wtab = jnp.zeros((B * (NV + 1),), jnp.int32).at[gidx].max(pid + 1)
    win = wtab[gidx] == pid + 1
    dummy = ((pid // HW) + 1) * (NV + 1) - 1
    gidx2 = jnp.where(win, gidx, dummy)

    vals = jnp.transpose(feature.reshape(B, C, HW), (0, 2, 1)).reshape(B * HW, C)
    vol_flat = jnp.zeros((B * (NV + 1), C), dtype=feature.dtype).at[gidx2].set(vals)
    volume = jnp.transpose(vol_flat.reshape(B, NV + 1, C)[:, :NV, :],
                           (0, 2, 1)).reshape(B, C, X, Y, Z)
    valid_flat = jnp.zeros((B * (NV + 1),), dtype=feature.dtype).at[gidx2].set(1.0)
    valid = valid_flat.reshape(B, NV + 1)[:, :NV].reshape(B, 1, X, Y, Z)
    return volume, valid


# speedup vs baseline: 4.1565x; 4.1565x over previous
import functools

import jax
import jax.numpy as jnp
from jax import lax
from jax.experimental import pallas as pl
from jax.experimental.pallas import tpu as pltpu
from jax.experimental.pallas import tpu_sc as plsc

X, Y, Z = 96, 96, 48
NV = X * Y * Z
B, C, H, W = 2, 32, 120, 160
HW = H * W
BHW = B * HW
BNV = B * NV
VS = 0.04
SENT = 2 ** 30
NW = 32
LSIZE = BNV // NW
CHUNK = 2400
NCH = BHW // CHUNK
TW = 128
VOLW = B * C * NV
IGN = -1


def _prep_body(d_ref, p_ref, o_ref):
    d = d_ref[...]
    hwi = lax.broadcasted_iota(jnp.int32, (B, HW), 1)
    u = (hwi % W).astype(jnp.float32)
    v = (hwi // W).astype(jnp.float32)

    def b16(x):
        return x.astype(jnp.bfloat16).astype(jnp.float32)

    ud = b16(u * d)
    vd = b16(v * d)
    db = b16(d)

    def wrow(i):
        m0 = b16(p_ref[:, 4 * i + 0:4 * i + 1])
        m1 = b16(p_ref[:, 4 * i + 1:4 * i + 2])
        m2 = b16(p_ref[:, 4 * i + 2:4 * i + 3])
        m3 = b16(p_ref[:, 4 * i + 3:4 * i + 4])
        return ((m0 * ud + m1 * vd) + m2 * db) + m3

    wx, wy, wz = wrow(0), wrow(1), wrow(2)
    vx = jnp.round((wx - p_ref[:, 12:13]) / VS).astype(jnp.int32)
    vy = jnp.round((wy - p_ref[:, 13:14]) / VS).astype(jnp.int32)
    vz = jnp.round((wz - p_ref[:, 14:15]) / VS).astype(jnp.int32)
    mask = ((d > 0) & (vx >= 0) & (vx < X) & (vy >= 0) & (vy < Y)
            & (vz >= 0) & (vz < Z))
    flat = vx * (Y * Z) + vy * Z + vz
    bb = lax.broadcasted_iota(jnp.int32, (B, HW), 0)
    o_ref[...] = jnp.where(mask, flat + bb * NV, SENT)


_prep = pl.pallas_call(
    _prep_body,
    out_shape=jax.ShapeDtypeStruct((B, HW), jnp.int32),
)


def _zero_body(o_ref):
    o_ref[...] = jnp.zeros_like(o_ref)


def _zeros(nrows, ncols, grid):
    return pl.pallas_call(
        _zero_body,
        grid=(grid,),
        out_specs=pl.BlockSpec((nrows // grid, ncols), lambda i: (i, 0)),
        out_shape=jax.ShapeDtypeStruct((nrows, ncols), jnp.float32),
    )


_zero_vol = _zeros(27648, 1024, 32)
_zero_valid = _zeros(864, 1024, 1)

_sc_mesh = plsc.VectorSubcoreMesh(core_axis_name="c", subcore_axis_name="s")


@functools.partial(
    pl.kernel,
    mesh=_sc_mesh,
    out_type=(),
    compiler_params=pltpu.CompilerParams(needs_layout_passes=False),
    scratch_types=[
        pltpu.VMEM((LSIZE,), jnp.int32),
        pltpu.VMEM((LSIZE + 16,), jnp.int32),
        pltpu.VMEM((CHUNK,), jnp.int32),
        pltpu.VMEM((TW * 32,), jnp.int32),
        pltpu.VMEM((TW * 32,), jnp.int32),
        pltpu.VMEM((TW * 32,), jnp.float32),
        pltpu.VMEM((TW,), jnp.int32),
        pltpu.VMEM((TW,), jnp.float32),
        pltpu.SemaphoreType.DMA,
        pltpu.SemaphoreType.DMA,
    ],
)
def _sc_scatter(gidx_hbm, feat_hbm, vol_ref, valid_ref,
                wt, comp, gbuf, sidx, didx, rows, vidx, ones, sem1, sem2):
    cid = lax.axis_index("c")
    sid = lax.axis_index("s")
    wid = sid * 2 + cid
    iota = lax.iota(jnp.int32, 16)

    def p0(k, _):
        wt[pl.ds(k * 16, 16)] = jnp.zeros((16,), jnp.int32)
        return 0
    lax.fori_loop(0, LSIZE // 16, p0, 0)

    def p0b(k, _):
        ones[pl.ds(k * 16, 16)] = jnp.ones((16,), jnp.float32)
        return 0
    lax.fori_loop(0, TW // 16, p0b, 0)

    def p1c(c, _):
        pltpu.sync_copy(gidx_hbm.at[pl.ds(c * CHUNK, CHUNK)], gbuf)

        def p1v(k, _):
            g = gbuf[pl.ds(k * 16, 16)]
            mine = ((g & 31) == wid) & (g < BNV)
            local = jnp.where(mine, g >> 5, 0)
            pid = c * CHUNK + k * 16 + iota
            plsc.store_scatter(wt, [local], pid + 1, mask=mine)
            return 0
        lax.fori_loop(0, CHUNK // 16, p1v, 0)
        return 0
    lax.fori_loop(0, NCH, p1c, 0)

    def p2(k, cnt):
        wv = wt[pl.ds(k * 16, 16)]
        m = wv > 0
        packed = wv | ((k * 16 + iota) << 16)
        plsc.store_compressed(comp.at[pl.ds(cnt, 16)], packed, mask=m)
        return cnt + jnp.max(plsc.all_reduce_population_count(m))
    cnt = lax.fori_loop(0, LSIZE // 16, p2, jnp.int32(0))

    ntile = (cnt + TW - 1) // TW

    def p3(t, _):
        def bld(r, _):
            i = t * TW + (r >> 1)
            ivec = jnp.full((16,), i, jnp.int32)
            ok = ivec < cnt
            pk = plsc.load_gather(comp, [jnp.where(ok, ivec, 0)])
            pid = (pk & 0xFFFF) - 1
            g = ((pk >> 16) << 5) + wid
            b = jnp.where(g >= NV, 1, 0)
            cvec = (r & 1) * 16 + iota
            src = (pid + b * ((C - 1) * HW)) + cvec * HW
            dst = (g + b * ((C - 1) * NV)) + cvec * NV
            sidx[pl.ds(r * 16, 16)] = jnp.where(ok, src, 0)
            didx[pl.ds(r * 16, 16)] = jnp.where(ok, dst, IGN)
            return 0
        lax.fori_loop(0, TW * 2, bld, 0)
        pltpu.async_copy(feat_hbm.at[sidx], rows, sem1).wait()
        pltpu.async_copy(
            rows, vol_ref.at[plsc.Indices(didx, ignored_value=IGN)], sem2
        ).wait()

        def vbld(q, _):
            ivec = t * TW + q * 16 + iota
            ok = ivec < cnt
            pk = plsc.load_gather(comp, [jnp.where(ok, ivec, 0)])
            g = ((pk >> 16) << 5) + wid
            vidx[pl.ds(q * 16, 16)] = jnp.where(ok, g, IGN)
            return 0
        lax.fori_loop(0, TW // 16, vbld, 0)
        pltpu.async_copy(
            ones, valid_ref.at[plsc.Indices(vidx, ignored_value=IGN)], sem2
        ).wait()
        return 0
    lax.fori_loop(0, ntile, p3, 0)


def kernel(projection, feature, depth, origin):
    bottom = jnp.broadcast_to(
        jnp.array([0.0, 0.0, 0.0, 1.0], dtype=projection.dtype).reshape(1, 1, 4),
        (B, 1, 4))
    inv = jnp.linalg.inv(jnp.concatenate([projection, bottom], axis=1))
    params = jnp.concatenate(
        [inv[:, 0, :], inv[:, 1, :], inv[:, 2, :], origin], axis=1)
    params = jnp.pad(params, ((0, 0), (0, 128 - params.shape[1])))

    gidx = _prep(depth.reshape(B, HW), params).reshape(BHW)
    feat = feature.reshape(-1)

    vol_ref = jax.new_ref(_zero_vol().reshape(-1))
    valid_ref = jax.new_ref(_zero_valid().reshape(-1))
    _sc_scatter(gidx, feat, vol_ref, valid_ref)
    volume = vol_ref[...].reshape(B, C, X, Y, Z)
    valid = valid_ref[...].reshape(B, 1, X, Y, Z)
    return volume, valid
